# baseline (speedup 1.0000x reference)
import jax
import jax.numpy as jnp
from jax import lax
from jax.experimental import pallas as pl
from jax.experimental.pallas import tpu as pltpu
from jax.experimental.pallas import tpu_sc as plsc

B = 8
N = 8192
G = 512
K = 32
GBLK = 256
LANES = 16


def _fps_body(x_hbm, y_hbm, z_hbm, f0_hbm, out_hbm,
              x_v, y_v, z_v, d_v, cx_v, cy_v, cz_v, f0_v):
    b = lax.axis_index("s") * 2 + lax.axis_index("c")

    @pl.when(b < B)
    def _():
        pltpu.sync_copy(x_hbm.at[pl.ds(b, 1)], x_v.at[:, pl.ds(0, N)])
        pltpu.sync_copy(y_hbm.at[pl.ds(b, 1)], y_v.at[:, pl.ds(0, N)])
        pltpu.sync_copy(z_hbm.at[pl.ds(b, 1)], z_v.at[:, pl.ds(0, N)])
        pltpu.sync_copy(f0_hbm, f0_v)

        def init(j, c):
            d_v[0, pl.ds(j * LANES, LANES)] = jnp.full((LANES,), 1e10,
                                                       jnp.float32)
            return c

        lax.fori_loop(0, N // LANES, init, 0, unroll=8)

        iota = lax.iota(jnp.int32, LANES)
        neg = jnp.full((LANES,), -3e38, jnp.float32)
        zero_i = jnp.zeros((LANES,), jnp.int32)

        def put_f(ref, base, lane, val):
            v = ref[0, pl.ds(base, LANES)]
            ref[0, pl.ds(base, LANES)] = jnp.where(iota == lane, val, v)

        def step(s, cur):
            curx = x_v[0, pl.ds(cur, LANES)][0]
            cury = y_v[0, pl.ds(cur, LANES)][0]
            curz = z_v[0, pl.ds(cur, LANES)][0]
            sbase = (s // LANES) * LANES
            slane = s - sbase
            put_f(cx_v, sbase, slane, curx)
            put_f(cy_v, sbase, slane, cury)
            put_f(cz_v, sbase, slane, curz)

            def chunks(i, acc):
                new = []
                for u in range(4):
                    bmax, bidx = acc[2 * u], acc[2 * u + 1]
                    off = i + u * LANES
                    sl = pl.ds(off, LANES)
                    dx = x_v[0, sl] - curx
                    dy = y_v[0, sl] - cury
                    dz = z_v[0, sl] - curz
                    dd = dx * dx + dy * dy + dz * dz
                    dmin = jnp.minimum(d_v[0, sl], dd)
                    d_v[0, sl] = dmin
                    upd = dmin > bmax
                    new.append(jnp.where(upd, dmin, bmax))
                    new.append(jnp.where(upd, iota + off, bidx))
                return tuple(new)

            acc = plsc.parallel_loop(0, N, step=4 * LANES, unroll=2,
                                     carry=(neg, zero_i) * 4)(chunks)

            def merge(a, b2):
                am, ai = a
                bm, bi = b2
                upd = (bm > am) | ((bm == am) & (bi < ai))
                return (jnp.where(upd, bm, am), jnp.where(upd, bi, ai))

            bmax, bidx = merge(merge((acc[0], acc[1]), (acc[2], acc[3])),
                               merge((acc[4], acc[5]), (acc[6], acc[7])))
            sk, _ = plsc.sort_key_val(bmax, bidx, descending=True)
            m = sk[0]
            cand = jnp.where(bmax == m, bidx, jnp.int32(N))
            sc, _ = plsc.sort_key_val(cand, cand)
            return sc[0]

        cur0 = f0_v[0, pl.ds(b, LANES)][0]
        lax.fori_loop(0, G, step, cur0)

        pltpu.sync_copy(cx_v, out_hbm.at[pl.ds(b * 3 + 0, 1)])
        pltpu.sync_copy(cy_v, out_hbm.at[pl.ds(b * 3 + 1, 1)])
        pltpu.sync_copy(cz_v, out_hbm.at[pl.ds(b * 3 + 2, 1)])


def _fps_centers(x, y, z, f0pad):
    mesh = plsc.VectorSubcoreMesh(core_axis_name="c", subcore_axis_name="s",
                                  num_cores=2, num_subcores=16)
    fps = pl.kernel(
        _fps_body,
        out_type=jax.ShapeDtypeStruct((B * 3, G), jnp.float32),
        mesh=mesh,
        scratch_types=[
            pltpu.VMEM((1, N + LANES), jnp.float32),
            pltpu.VMEM((1, N + LANES), jnp.float32),
            pltpu.VMEM((1, N + LANES), jnp.float32),
            pltpu.VMEM((1, N), jnp.float32),
            pltpu.VMEM((1, G), jnp.float32),
            pltpu.VMEM((1, G), jnp.float32),
            pltpu.VMEM((1, G), jnp.float32),
            pltpu.VMEM((1, 2 * LANES), jnp.int32),
        ],
        compiler_params=pltpu.CompilerParams(needs_layout_passes=False),
    )
    return fps(x, y, z, f0pad)


def _group_body(xyz_ref, c_ref, o_ref):
    x = xyz_ref[0, 0:1, :]
    y = xyz_ref[0, 1:2, :]
    z = xyz_ref[0, 2:3, :]
    cx = c_ref[0, :, 0:1]
    cy = c_ref[0, :, 1:2]
    cz = c_ref[0, :, 2:3]

    cmat = c_ref[0]
    xmat = xyz_ref[0]
    prod = jax.lax.dot_general(cmat, xmat, (((1,), (0,)), ((), ())),
                               preferred_element_type=jnp.float32)
    c2 = cx * cx + cy * cy + cz * cz
    p2 = x * x + y * y + z * z
    d = (-2.0 * prod + c2) + p2
    d = jnp.maximum(d, 0.0)
    k = lax.bitcast_convert_type(d, jnp.int32)

    def bs(_, lohi):
        lo, hi = lohi
        mid = lo + lax.shift_right_logical(hi - lo, 1)
        cnt = jnp.sum((k <= mid).astype(jnp.int32), axis=1, keepdims=True)
        ge = cnt >= K
        return (jnp.where(ge, lo, mid + 1), jnp.where(ge, mid, hi))

    lo0 = jnp.zeros((GBLK, 1), jnp.int32)
    hi0 = jnp.full((GBLK, 1), jnp.int32(0x7F7FFFFF))
    _, t = lax.fori_loop(0, 31, bs, (lo0, hi0))

    lt = k < t
    eq = k == t
    c_lt = jnp.sum(lt.astype(jnp.int32), axis=1, keepdims=True)
    m = jnp.sum(eq.astype(jnp.int32), axis=1, keepdims=True)
    w_eq = (K - c_lt).astype(jnp.float32) / m.astype(jnp.float32)
    w = lt.astype(jnp.float32) + w_eq * eq.astype(jnp.float32)

    scale = jnp.float32(1.0 / K)
    o_ref[0, :, 0:1] = jnp.sum(w * x, axis=1, keepdims=True) * scale
    o_ref[0, :, 1:2] = jnp.sum(w * y, axis=1, keepdims=True) * scale
    o_ref[0, :, 2:3] = jnp.sum(w * z, axis=1, keepdims=True) * scale


def _group_mean(xyzT, center):
    return pl.pallas_call(
        _group_body,
        out_shape=jax.ShapeDtypeStruct((B, G, 3), jnp.float32),
        grid=(B, G // GBLK),
        in_specs=[
            pl.BlockSpec((1, 3, N), lambda b, g: (b, 0, 0)),
            pl.BlockSpec((1, GBLK, 3), lambda b, g: (b, g, 0)),
        ],
        out_specs=pl.BlockSpec((1, GBLK, 3), lambda b, g: (b, g, 0)),
        compiler_params=pltpu.CompilerParams(
            dimension_semantics=("parallel", "parallel"),
        ),
    )(xyzT, center)


def kernel(xyz):
    xyzT = jnp.transpose(xyz, (0, 2, 1))
    x = xyzT[:, 0]
    y = xyzT[:, 1]
    z = xyzT[:, 2]
    f0 = jax.random.randint(jax.random.key(1), (B,), 0, N).astype(jnp.int32)
    f0pad = jnp.zeros((1, 2 * LANES), jnp.int32).at[0, :B].set(f0)
    centersT = _fps_centers(x, y, z, f0pad)
    center = jnp.transpose(centersT.reshape(B, 3, G), (0, 2, 1))
    gf = _group_mean(xyzT, center)
    return (center, gf)


# speedup vs baseline: 1.0149x; 1.0149x over previous
import jax
import jax.numpy as jnp
from jax import lax
from jax.experimental import pallas as pl
from jax.experimental.pallas import tpu as pltpu
from jax.experimental.pallas import tpu_sc as plsc

B = 8
N = 8192
G = 512
K = 32
GBLK = 512
LANES = 16


def _fps_body(x_hbm, y_hbm, z_hbm, f0_hbm, out_hbm,
              x_v, y_v, z_v, d_v, cx_v, cy_v, cz_v, f0_v):
    b = lax.axis_index("s") * 2 + lax.axis_index("c")

    @pl.when(b < B)
    def _():
        pltpu.sync_copy(x_hbm.at[pl.ds(b, 1)], x_v.at[:, pl.ds(0, N)])
        pltpu.sync_copy(y_hbm.at[pl.ds(b, 1)], y_v.at[:, pl.ds(0, N)])
        pltpu.sync_copy(z_hbm.at[pl.ds(b, 1)], z_v.at[:, pl.ds(0, N)])
        pltpu.sync_copy(f0_hbm, f0_v)

        def init(j, c):
            d_v[0, pl.ds(j * LANES, LANES)] = jnp.full((LANES,), 1e10,
                                                       jnp.float32)
            return c

        lax.fori_loop(0, N // LANES, init, 0, unroll=8)

        iota = lax.iota(jnp.int32, LANES)
        neg = jnp.full((LANES,), -3e38, jnp.float32)
        zero_i = jnp.zeros((LANES,), jnp.int32)

        def put_f(ref, base, lane, val):
            v = ref[0, pl.ds(base, LANES)]
            ref[0, pl.ds(base, LANES)] = jnp.where(iota == lane, val, v)

        def step(s, cur):
            curx = x_v[0, pl.ds(cur, LANES)][0]
            cury = y_v[0, pl.ds(cur, LANES)][0]
            curz = z_v[0, pl.ds(cur, LANES)][0]
            sbase = (s // LANES) * LANES
            slane = s - sbase
            put_f(cx_v, sbase, slane, curx)
            put_f(cy_v, sbase, slane, cury)
            put_f(cz_v, sbase, slane, curz)

            def chunks(i, acc):
                new = []
                for u in range(4):
                    bmax, bidx = acc[2 * u], acc[2 * u + 1]
                    off = i + u * LANES
                    sl = pl.ds(off, LANES)
                    dx = x_v[0, sl] - curx
                    dy = y_v[0, sl] - cury
                    dz = z_v[0, sl] - curz
                    dd = dx * dx + dy * dy + dz * dz
                    dmin = jnp.minimum(d_v[0, sl], dd)
                    d_v[0, sl] = dmin
                    upd = dmin > bmax
                    new.append(jnp.where(upd, dmin, bmax))
                    new.append(jnp.where(upd, iota + off, bidx))
                return tuple(new)

            acc = plsc.parallel_loop(0, N, step=4 * LANES, unroll=2,
                                     carry=(neg, zero_i) * 4)(chunks)

            def merge(a, b2):
                am, ai = a
                bm, bi = b2
                upd = (bm > am) | ((bm == am) & (bi < ai))
                return (jnp.where(upd, bm, am), jnp.where(upd, bi, ai))

            bmax, bidx = merge(merge((acc[0], acc[1]), (acc[2], acc[3])),
                               merge((acc[4], acc[5]), (acc[6], acc[7])))
            sk, _ = plsc.sort_key_val(bmax, bidx, descending=True)
            m = sk[0]
            cand = jnp.where(bmax == m, bidx, jnp.int32(N))
            sc, _ = plsc.sort_key_val(cand, cand)
            return sc[0]

        cur0 = f0_v[0, pl.ds(b, LANES)][0]
        lax.fori_loop(0, G, step, cur0)

        pltpu.sync_copy(cx_v, out_hbm.at[pl.ds(b * 3 + 0, 1)])
        pltpu.sync_copy(cy_v, out_hbm.at[pl.ds(b * 3 + 1, 1)])
        pltpu.sync_copy(cz_v, out_hbm.at[pl.ds(b * 3 + 2, 1)])


def _fps_centers(x, y, z, f0pad):
    mesh = plsc.VectorSubcoreMesh(core_axis_name="c", subcore_axis_name="s",
                                  num_cores=2, num_subcores=16)
    fps = pl.kernel(
        _fps_body,
        out_type=jax.ShapeDtypeStruct((B * 3, G), jnp.float32),
        mesh=mesh,
        scratch_types=[
            pltpu.VMEM((1, N + LANES), jnp.float32),
            pltpu.VMEM((1, N + LANES), jnp.float32),
            pltpu.VMEM((1, N + LANES), jnp.float32),
            pltpu.VMEM((1, N), jnp.float32),
            pltpu.VMEM((1, G), jnp.float32),
            pltpu.VMEM((1, G), jnp.float32),
            pltpu.VMEM((1, G), jnp.float32),
            pltpu.VMEM((1, 2 * LANES), jnp.int32),
        ],
        compiler_params=pltpu.CompilerParams(needs_layout_passes=False),
    )
    return fps(x, y, z, f0pad)


def _group_body(xyz_ref, c_ref, o_ref):
    x = xyz_ref[0, 0:1, :]
    y = xyz_ref[0, 1:2, :]
    z = xyz_ref[0, 2:3, :]
    cx = c_ref[0, :, 0:1]
    cy = c_ref[0, :, 1:2]
    cz = c_ref[0, :, 2:3]

    cmat = c_ref[0]
    xmat = xyz_ref[0]
    prod = jax.lax.dot_general(cmat, xmat, (((1,), (0,)), ((), ())),
                               preferred_element_type=jnp.float32)
    c2 = cx * cx + cy * cy + cz * cz
    p2 = x * x + y * y + z * z
    d = (-2.0 * prod + c2) + p2
    d = jnp.maximum(d, 0.0)
    k = lax.bitcast_convert_type(d, jnp.int32)

    def bs(_, lohi):
        lo, hi = lohi
        mid = lo + lax.shift_right_logical(hi - lo, 1)
        cnt = jnp.sum((k <= mid).astype(jnp.int32), axis=1, keepdims=True)
        ge = cnt >= K
        return (jnp.where(ge, lo, mid + 1), jnp.where(ge, mid, hi))

    lo0 = jnp.zeros((GBLK, 1), jnp.int32)
    hi0 = jnp.full((GBLK, 1), jnp.int32(0x7F7FFFFF))
    _, t = lax.fori_loop(0, 31, bs, (lo0, hi0))

    lt = k < t
    eq = k == t
    c_lt = jnp.sum(lt.astype(jnp.int32), axis=1, keepdims=True)
    m = jnp.sum(eq.astype(jnp.int32), axis=1, keepdims=True)
    w_eq = (K - c_lt).astype(jnp.float32) / m.astype(jnp.float32)
    w = lt.astype(jnp.float32) + w_eq * eq.astype(jnp.float32)

    scale = jnp.float32(1.0 / K)
    o_ref[0, :, 0:1] = jnp.sum(w * x, axis=1, keepdims=True) * scale
    o_ref[0, :, 1:2] = jnp.sum(w * y, axis=1, keepdims=True) * scale
    o_ref[0, :, 2:3] = jnp.sum(w * z, axis=1, keepdims=True) * scale


def _group_mean(xyzT, center):
    return pl.pallas_call(
        _group_body,
        out_shape=jax.ShapeDtypeStruct((B, G, 3), jnp.float32),
        grid=(B, G // GBLK),
        in_specs=[
            pl.BlockSpec((1, 3, N), lambda b, g: (b, 0, 0)),
            pl.BlockSpec((1, GBLK, 3), lambda b, g: (b, g, 0)),
        ],
        out_specs=pl.BlockSpec((1, GBLK, 3), lambda b, g: (b, g, 0)),
        compiler_params=pltpu.CompilerParams(
            dimension_semantics=("parallel", "parallel"),
        ),
    )(xyzT, center)


def kernel(xyz):
    xyzT = jnp.transpose(xyz, (0, 2, 1))
    x = xyzT[:, 0]
    y = xyzT[:, 1]
    z = xyzT[:, 2]
    f0 = jax.random.randint(jax.random.key(1), (B,), 0, N).astype(jnp.int32)
    f0pad = jnp.zeros((1, 2 * LANES), jnp.int32).at[0, :B].set(f0)
    centersT = _fps_centers(x, y, z, f0pad)
    center = jnp.transpose(centersT.reshape(B, 3, G), (0, 2, 1))
    gf = _group_mean(xyzT, center)
    return (center, gf)
